# baseline (speedup 1.0000x reference)
import functools
import math

import jax
import jax.numpy as jnp
from jax import lax
from jax.experimental import pallas as pl
from jax.experimental.pallas import tpu as pltpu
from jax.experimental.pallas import tpu_sc as plsc

_LANES = 16


def _pool_sc(idx_flat, emb_flat, batch, ctx, d, v):
    info = plsc.get_sparse_core_info()
    nw = info.num_cores * info.num_subcores
    rows_per_w = d // nw if d >= nw else 1
    n_chunks = batch // _LANES

    mesh = plsc.VectorSubcoreMesh(core_axis_name="c", subcore_axis_name="s")

    @functools.partial(
        pl.kernel,
        mesh=mesh,
        out_type=jax.ShapeDtypeStruct((d * batch,), jnp.float32),
        scratch_types=[
            pltpu.VMEM((ctx * batch,), jnp.int32),
            pltpu.VMEM((v,), jnp.float32),
            pltpu.VMEM((rows_per_w * batch,), jnp.float32),
            pltpu.SemaphoreType.DMA,
        ],
        compiler_params=pltpu.CompilerParams(
            use_tc_tiling_on_sc=False, needs_layout_passes=False
        ),
    )
    def pool(idx_hbm, emb_hbm, out_hbm, idx_v, row_v, acc_v, sem):
        wid = lax.axis_index("s") * info.num_cores + lax.axis_index("c")
        pltpu.sync_copy(idx_hbm, idx_v)
        for i in range(rows_per_w):
            r = wid * rows_per_w + i
            pltpu.async_copy(emb_hbm.at[pl.ds(r * v, v)], row_v, sem).wait()

            def body(k, carry):
                acc = plsc.load_gather(row_v, [idx_v[pl.ds(k * _LANES, _LANES)]])
                for c in range(1, ctx):
                    acc = acc + plsc.load_gather(
                        row_v, [idx_v[pl.ds(c * batch + k * _LANES, _LANES)]]
                    )
                acc_v[pl.ds(i * batch + k * _LANES, _LANES)] = acc
                return carry

            lax.fori_loop(0, n_chunks, body, 0)
        pltpu.sync_copy(
            acc_v, out_hbm.at[pl.ds(wid * rows_per_w * batch, rows_per_w * batch)]
        )

    return pool(idx_flat, emb_flat)


def _mm_body(xt_ref, wt_ref, b_ref, o_ref):
    o_ref[...] = (
        lax.dot_general(
            wt_ref[...], xt_ref[...],
            (((0,), (0,)), ((), ())),
            preferred_element_type=jnp.float32,
        )
        + b_ref[0, 0][:, None]
    )


def _fc1_tc(x_sum_t, fc1_w, fc1_b, vb=4096):
    d, batch = x_sum_t.shape
    v = fc1_w.shape[0]
    nb = math.ceil(v / vb)
    wt = fc1_w.T
    b_pad = jnp.zeros((nb * vb,), jnp.float32).at[:v].set(fc1_b)
    return pl.pallas_call(
        _mm_body,
        grid=(nb,),
        in_specs=[
            pl.BlockSpec((d, batch), lambda j: (0, 0)),
            pl.BlockSpec((d, vb), lambda j: (0, j)),
            pl.BlockSpec((1, 1, vb), lambda j: (j, 0, 0)),
        ],
        out_specs=pl.BlockSpec((vb, batch), lambda j: (j, 0)),
        out_shape=jax.ShapeDtypeStruct((v, batch), jnp.float32),
        compiler_params=pltpu.CompilerParams(
            dimension_semantics=("arbitrary",),
        ),
    )(x_sum_t, wt, b_pad.reshape(nb, 1, vb))


def kernel(x_in, embedding, fc1_w, fc1_b):
    batch, ctx = x_in.shape
    v, d = embedding.shape
    idx_flat = x_in.astype(jnp.int32).T.reshape(-1)
    dh = d // 2
    halves = []
    for h in range(2):
        emb_flat = embedding[:, h * dh:(h + 1) * dh].T.reshape(-1)
        halves.append(
            _pool_sc(idx_flat, emb_flat, batch, ctx, dh, v).reshape(dh, batch)
        )
    x_sum_t = jnp.concatenate(halves, axis=0)
    return _fc1_tc(x_sum_t, fc1_w, fc1_b).T


# speedup vs baseline: 1.0689x; 1.0689x over previous
import functools
import math

import jax
import jax.numpy as jnp
from jax import lax
from jax.experimental import pallas as pl
from jax.experimental.pallas import tpu as pltpu
from jax.experimental.pallas import tpu_sc as plsc

_LANES = 16


def _pool_sc(idx_flat, emb_flat, batch, ctx, d, v):
    info = plsc.get_sparse_core_info()
    nw = info.num_cores * info.num_subcores
    rows_per_w = d // nw if d >= nw else 1
    n_chunks = batch // _LANES

    mesh = plsc.VectorSubcoreMesh(core_axis_name="c", subcore_axis_name="s")

    @functools.partial(
        pl.kernel,
        mesh=mesh,
        out_type=jax.ShapeDtypeStruct((d * batch,), jnp.float32),
        scratch_types=[
            pltpu.VMEM((ctx * batch,), jnp.int32),
            pltpu.VMEM((v,), jnp.float32),
            pltpu.VMEM((rows_per_w * batch,), jnp.float32),
            pltpu.SemaphoreType.DMA,
        ],
        compiler_params=pltpu.CompilerParams(
            use_tc_tiling_on_sc=False, needs_layout_passes=False
        ),
    )
    def pool(idx_hbm, emb_hbm, out_hbm, idx_v, row_v, acc_v, sem):
        wid = lax.axis_index("s") * info.num_cores + lax.axis_index("c")
        pltpu.sync_copy(idx_hbm, idx_v)
        for i in range(rows_per_w):
            r = wid * rows_per_w + i
            pltpu.async_copy(emb_hbm.at[pl.ds(r * v, v)], row_v, sem).wait()

            def body(k, carry):
                acc = plsc.load_gather(row_v, [idx_v[pl.ds(k * _LANES, _LANES)]])
                for c in range(1, ctx):
                    acc = acc + plsc.load_gather(
                        row_v, [idx_v[pl.ds(c * batch + k * _LANES, _LANES)]]
                    )
                acc_v[pl.ds(i * batch + k * _LANES, _LANES)] = acc
                return carry

            lax.fori_loop(0, n_chunks, body, 0)
        pltpu.sync_copy(
            acc_v, out_hbm.at[pl.ds(wid * rows_per_w * batch, rows_per_w * batch)]
        )

    return pool(idx_flat, emb_flat)


def _mm_body(xt_ref, wt_ref, b_ref, o_ref):
    o_ref[...] = (
        lax.dot_general(
            wt_ref[...], xt_ref[...],
            (((0,), (0,)), ((), ())),
            preferred_element_type=jnp.float32,
        )
        + b_ref[0, 0][:, None]
    )


def _fc1_tc(x_sum_t, fc1_w, fc1_b, vb=4096):
    d, batch = x_sum_t.shape
    v = fc1_w.shape[0]
    nb = math.ceil(v / vb)
    wt = fc1_w.T
    b_pad = jnp.zeros((nb * vb,), jnp.float32).at[:v].set(fc1_b)
    return pl.pallas_call(
        _mm_body,
        grid=(nb,),
        in_specs=[
            pl.BlockSpec((d, batch), lambda j: (0, 0)),
            pl.BlockSpec((d, vb), lambda j: (0, j)),
            pl.BlockSpec((1, 1, vb), lambda j: (j, 0, 0)),
        ],
        out_specs=pl.BlockSpec((vb, batch), lambda j: (j, 0)),
        out_shape=jax.ShapeDtypeStruct((v, batch), jnp.float32),
        compiler_params=pltpu.CompilerParams(
            dimension_semantics=("arbitrary",),
        ),
    )(x_sum_t, wt, b_pad.reshape(nb, 1, vb))


def kernel(x_in, embedding, fc1_w, fc1_b):
    batch, ctx = x_in.shape
    v, d = embedding.shape
    idx_flat = x_in.astype(jnp.int32).T.reshape(-1)
    emb_flat = embedding.T.reshape(-1)
    x_sum_t = _pool_sc(idx_flat, emb_flat, batch, ctx, d, v).reshape(d, batch)
    return _fc1_tc(x_sum_t, fc1_w, fc1_b).T
